# Initial kernel scaffold; baseline (speedup 1.0000x reference)
import jax
import jax.numpy as jnp
from jax.experimental import pallas as pl


def kernel(x, edge_index, mmse, Wl1, Wr1, a1, b1, Wl2, Wr2, a2, b2, Wm, bm, W11, b11, W12, b12, W21, b21, W22, b22, Wp, bp):
    raise NotImplementedError("write your pallas kernel here")


# speedup vs baseline: 14.9502x; 14.9502x over previous
import jax
import jax.numpy as jnp
from jax.experimental import pallas as pl

N = 19
E = 342
_PREC = jax.lax.Precision.HIGHEST


def _fused(x_ref, ei_ref, mmse_ref, wl1_ref, wr1_ref, a1_ref, b1_ref,
           wl2_ref, wr2_ref, a2_ref, b2_ref, wm_ref, bm_ref,
           comp_ref, alpha_ref):
    f32 = jnp.float32
    src = ei_ref[0:1, :]
    dst = ei_ref[1:2, :]
    iota_ne = jax.lax.broadcasted_iota(jnp.int32, (N, E), 0)
    s_oh = (iota_ne == src).astype(f32)
    d_oh = (iota_ne == dst).astype(f32)
    c2 = jax.lax.dot_general(s_oh, d_oh, (((1,), (1,)), ((), ())),
                             precision=_PREC, preferred_element_type=f32)
    has = c2 > 0.0

    def gatv2(h, wl, wr, a, b):
        xl = jnp.dot(h, wl, precision=_PREC, preferred_element_type=f32)
        xr = jnp.dot(h, wr, precision=_PREC, preferred_element_type=f32)
        z = xl[:, None, :] + xr[None, :, :]
        lz = jnp.where(z > 0, z, 0.2 * z)
        e2 = jnp.sum(lz * a.reshape(1, 1, -1), axis=2)
        m = jnp.max(jnp.where(has, e2, -1e30), axis=0, keepdims=True)
        ex = jnp.where(has, jnp.exp(e2 - m), 0.0)
        ssum = jnp.sum(c2 * ex, axis=0, keepdims=True)
        alpha = ex / (ssum + 1e-16)
        wmat = c2 * alpha
        out = jax.lax.dot_general(wmat, xl, (((0,), (0,)), ((), ())),
                                  precision=_PREC, preferred_element_type=f32)
        return out + b, alpha

    h1, alpha1 = gatv2(x_ref[...], wl1_ref[...], wr1_ref[...],
                       a1_ref[...], b1_ref[...])
    h2, _ = gatv2(h1, wl2_ref[...], wr2_ref[...], a2_ref[...], b2_ref[...])
    gf = h2 + mmse_ref[...] * wm_ref[...] + bm_ref[...]
    dec = jax.lax.dot_general(gf, gf, (((1,), (1,)), ((), ())),
                              precision=_PREC, preferred_element_type=f32)
    comp_ref[...] = jax.nn.sigmoid(dec)
    u = jax.lax.dot_general(alpha1, d_oh, (((1,), (0,)), ((), ())),
                            precision=_PREC, preferred_element_type=f32)
    alpha_ref[...] = jnp.sum(s_oh * u, axis=0, keepdims=True)


def kernel(x, edge_index, mmse, Wl1, Wr1, a1, b1, Wl2, Wr2, a2, b2, Wm, bm,
           W11, b11, W12, b12, W21, b21, W22, b22, Wp, bp):
    compressed, alpha_2d = pl.pallas_call(
        _fused,
        out_shape=[
            jax.ShapeDtypeStruct((N, N), jnp.float32),
            jax.ShapeDtypeStruct((1, E), jnp.float32),
        ],
    )(x, edge_index, mmse.reshape(1, 1),
      Wl1, Wr1, a1.reshape(1, -1), b1.reshape(1, -1),
      Wl2, Wr2, a2.reshape(1, -1), b2.reshape(1, -1),
      Wm, bm.reshape(1, -1))
    return compressed, alpha_2d.reshape(E)
